# baseline (speedup 1.0000x reference)
import functools

import jax
import jax.numpy as jnp
from jax import lax
from jax.experimental import pallas as pl
from jax.experimental.pallas import tpu as pltpu
from jax.experimental.pallas import tpu_sc as plsc

B, L, D, V = 32, 2048, 768, 6561
NW = 32
NB = 4
BC = B // NB
TOK_PER_W = (BC * L) // NW
CHUNK = 128
NCHUNK = TOK_PER_W // CHUNK


def _sc_gather(codebook, idx_flat):
    mesh = plsc.VectorSubcoreMesh(core_axis_name="c", subcore_axis_name="s")

    @functools.partial(
        pl.kernel,
        mesh=mesh,
        out_type=jax.ShapeDtypeStruct((BC * L, D), jnp.float32),
        scratch_types=[
            pltpu.VMEM((CHUNK,), jnp.int32),
            pltpu.VMEM((CHUNK, D), jnp.float32),
            pltpu.SemaphoreType.DMA,
        ],
    )
    def k(table_hbm, idx_hbm, out_hbm, idx_v, rows_v, sem):
        wid = lax.axis_index("s") * 2 + lax.axis_index("c")
        base = wid * TOK_PER_W
        for c in range(NCHUNK):
            off = base + c * CHUNK
            pltpu.sync_copy(idx_hbm.at[pl.ds(off, CHUNK)], idx_v)
            pltpu.async_copy(table_hbm.at[idx_v], rows_v, sem).wait()
            pltpu.sync_copy(rows_v, out_hbm.at[pl.ds(off, CHUNK)])

    return k(codebook, idx_flat)


def _transpose_first_body(x_ref, o_ref):
    o_ref[...] = jnp.transpose(x_ref[...], (0, 2, 1))


def _transpose_acc_body(acc_ref, x_ref, o_ref):
    del acc_ref
    o_ref[...] = jnp.transpose(x_ref[...], (0, 2, 1))


def _tc_transpose_chunk(acc, rows, cidx):
    out_spec = pl.BlockSpec((1, D, L), lambda b: (cidx * BC + b, 0, 0))
    in_rows_spec = pl.BlockSpec((1, L, D), lambda b: (b, 0, 0))
    if acc is None:
        return pl.pallas_call(
            _transpose_first_body,
            grid=(BC,),
            in_specs=[in_rows_spec],
            out_specs=out_spec,
            out_shape=jax.ShapeDtypeStruct((B, D, L), jnp.float32),
        )(rows)
    return pl.pallas_call(
        _transpose_acc_body,
        grid=(BC,),
        in_specs=[pl.BlockSpec(memory_space=pl.ANY), in_rows_spec],
        out_specs=out_spec,
        out_shape=jax.ShapeDtypeStruct((B, D, L), jnp.float32),
        input_output_aliases={0: 0},
    )(acc, rows)


def kernel(audio, speech_token, codebook):
    idx_flat = speech_token.reshape(-1).astype(jnp.int32)
    chunk_rows = [
        _sc_gather(codebook, idx_flat[c * (BC * L):(c + 1) * (BC * L)])
        .reshape(BC, L, D)
        for c in range(NB)
    ]
    acc = None
    for c in range(NB):
        acc = _tc_transpose_chunk(acc, chunk_rows[c], c)
    return acc
